# Initial kernel scaffold; baseline (speedup 1.0000x reference)
import jax
import jax.numpy as jnp
from jax.experimental import pallas as pl


def kernel(x, weight):
    raise NotImplementedError("write your pallas kernel here")


# speedup vs baseline: 1.5742x; 1.5742x over previous
import functools

import jax
import jax.numpy as jnp
from jax import lax
from jax.experimental import pallas as pl
from jax.experimental.pallas import tpu as pltpu
from jax.experimental.pallas import tpu_sc as plsc

N_VOCAB = 1000000
N_EMBED = 64
N_TOKENS = 16384 * 50

NC = 2
NS = 16
NW = NC * NS

G = 128
PER_W = N_TOKENS // NW
N_GROUPS = PER_W // G

_mesh = plsc.VectorSubcoreMesh(
    core_axis_name="c", subcore_axis_name="s", num_cores=NC, num_subcores=NS
)


@functools.partial(
    pl.kernel,
    mesh=_mesh,
    compiler_params=pltpu.CompilerParams(use_tc_tiling_on_sc=False),
    out_type=jax.ShapeDtypeStruct((N_TOKENS, N_EMBED), jnp.float32),
    scratch_types=[
        pltpu.VMEM((G,), jnp.int32),
        pltpu.VMEM((G, N_EMBED), jnp.float32),
        pltpu.SemaphoreType.DMA,
    ],
)
def _emb_lookup(idx_hbm, table_hbm, out_hbm, idx_v, rows_v, sem):
    wid = lax.axis_index("s") * NC + lax.axis_index("c")
    base = wid * PER_W

    def body(g, carry):
        off = base + g * G
        pltpu.sync_copy(idx_hbm.at[pl.ds(off, G)], idx_v)
        pltpu.async_copy(table_hbm.at[idx_v], rows_v, sem).wait()
        pltpu.sync_copy(rows_v, out_hbm.at[pl.ds(off, G)])
        return carry

    lax.fori_loop(0, N_GROUPS, body, 0)


def kernel(x, weight):
    flat = x.reshape(-1).astype(jnp.int32)
    out = _emb_lookup(flat, weight)
    return out.reshape(x.shape + (weight.shape[1],))


# speedup vs baseline: 1.8765x; 1.1920x over previous
import functools

import jax
import jax.numpy as jnp
from jax import lax
from jax.experimental import pallas as pl
from jax.experimental.pallas import tpu as pltpu
from jax.experimental.pallas import tpu_sc as plsc

N_VOCAB = 1000000
N_EMBED = 64
N_TOKENS = 16384 * 50

NC = 2
NS = 16
NW = NC * NS

G = 128
N_GROUPS = N_TOKENS // (NW * G)
K = 5
N_CHUNKS = N_GROUPS // K

_mesh = plsc.VectorSubcoreMesh(
    core_axis_name="c", subcore_axis_name="s", num_cores=NC, num_subcores=NS
)


@functools.partial(
    pl.kernel,
    mesh=_mesh,
    compiler_params=pltpu.CompilerParams(use_tc_tiling_on_sc=False),
    out_type=jax.ShapeDtypeStruct((N_TOKENS // G, G, N_EMBED), jnp.float32),
    scratch_types=[
        pltpu.VMEM((N_GROUPS, G), jnp.int32),
        pltpu.VMEM((K, G, N_EMBED), jnp.float32),
        pltpu.VMEM((K, G, N_EMBED), jnp.float32),
        pltpu.SemaphoreType.DMA,
        pltpu.SemaphoreType.DMA,
        pltpu.SemaphoreType.DMA,
        pltpu.SemaphoreType.DMA,
    ],
)
def _emb_lookup(idx_hbm, table_hbm, out_hbm, idx_all, rows0, rows1,
                semg0, semg1, sems0, sems1):
    wid = lax.axis_index("s") * NC + lax.axis_index("c")
    gbase = wid * N_GROUPS

    pltpu.sync_copy(idx_hbm.at[pl.ds(gbase, N_GROUPS)], idx_all)

    rows = (rows0, rows1)
    semg = (semg0, semg1)
    sems = (sems0, sems1)

    def fire_gathers(c, b):
        for j in range(K):
            pltpu.async_copy(
                table_hbm.at[idx_all.at[c * K + j]], rows[b].at[j], semg[b]
            )

    def drain_gathers(b):
        pltpu.make_async_copy(out_hbm.at[pl.ds(0, K)], rows[b], semg[b]).wait()

    def fire_store(c, b):
        pltpu.async_copy(rows[b], out_hbm.at[pl.ds(gbase + c * K, K)], sems[b])

    def drain_store(b):
        pltpu.make_async_copy(rows[b], out_hbm.at[pl.ds(0, K)], sems[b]).wait()

    fire_gathers(0, 0)

    def pair_body(p, carry):
        c0 = 2 * p

        @pl.when(p >= 1)
        def _():
            drain_store(1)

        fire_gathers(c0 + 1, 1)
        drain_gathers(0)
        fire_store(c0, 0)

        @pl.when(p < N_CHUNKS // 2 - 1)
        def _():
            drain_store(0)
            fire_gathers(c0 + 2, 0)

        drain_gathers(1)
        fire_store(c0 + 1, 1)
        return carry

    lax.fori_loop(0, N_CHUNKS // 2, pair_body, 0)

    drain_store(0)
    drain_store(1)


def kernel(x, weight):
    idx2d = x.reshape(N_TOKENS // G, G).astype(jnp.int32)
    out = _emb_lookup(idx2d, weight)
    return out.reshape(x.shape + (weight.shape[1],))
